# Initial kernel scaffold; baseline (speedup 1.0000x reference)
import functools

import jax
import jax.numpy as jnp
from jax import lax
from jax.experimental import pallas as pl
from jax.experimental.pallas import tpu as pltpu
from jax.experimental.pallas import tpu_sc as plsc

N = 10000
E = 320000
D = 128
H = 128
L = 4
C = 32
G = 256

NC = 2
NS = 16
EPC = E // NC
EPT = EPC // NS
K = 128
NFULL = EPT // K
TAIL = EPT - NFULL * K
RPT = N // NS
ZR = 25

PR_T = (N // NC) // NS
PK = 104
PNC = PR_T // PK
PREM = N // NC - NS * PR_T
GPT = G // NS


def _zero_vmem(ref, rows, cols):
    zero16 = jnp.zeros((16,), jnp.float32)
    for r in range(rows):
        for q in range(cols // 16):
            ref[r, pl.ds(q * 16, 16)] = zero16


def _agg_body(with_deg, *refs):
    if with_deg:
        (h_hbm, src_hbm, dst_hbm, out_hbm, deg_hbm,
         acc, dega, sidx, didx, rows, sidx_t, didx_t, rows_t,
         zbuf, zbufd, ones_v, ones_t, sem) = refs
    else:
        (h_hbm, src_hbm, dst_hbm, out_hbm,
         acc, sidx, didx, rows, sidx_t, didx_t, rows_t, zbuf, sem) = refs

    c = lax.axis_index("c")
    s = lax.axis_index("s")

    _zero_vmem(zbuf, ZR, D)
    rbase = s * RPT

    @pl.loop(0, RPT // ZR)
    def _(i):
        pltpu.sync_copy(zbuf, acc.at[pl.ds(rbase + i * ZR, ZR), :])

    if with_deg:
        _zero_vmem(zbufd, ZR, 16)
        for r in range(K):
            ones_v[r, :] = jnp.full((16,), 1.0, jnp.float32)
        ones_t[...] = jnp.ones((16,), jnp.float32)

        @pl.loop(0, RPT // ZR)
        def _(i):
            pltpu.sync_copy(zbufd, dega.at[pl.ds(rbase + i * ZR, ZR), :])

    plsc.subcore_barrier()

    ebase = c * EPC + s * EPT

    @pl.loop(0, NFULL)
    def _(i):
        off = pl.multiple_of(ebase + i * K, 8)
        pltpu.sync_copy(src_hbm.at[pl.ds(off, K)], sidx)
        pltpu.sync_copy(dst_hbm.at[pl.ds(off, K)], didx)
        pltpu.async_copy(h_hbm.at[sidx], rows, sem).wait()
        pltpu.sync_copy(rows, acc.at[didx], add=True)
        if with_deg:
            pltpu.sync_copy(ones_v, dega.at[didx], add=True)

    if TAIL:
        off = pl.multiple_of(ebase + NFULL * K, 8)
        pltpu.sync_copy(src_hbm.at[pl.ds(off, TAIL)], sidx_t)
        pltpu.sync_copy(dst_hbm.at[pl.ds(off, TAIL)], didx_t)
        pltpu.async_copy(h_hbm.at[sidx_t], rows_t, sem).wait()
        pltpu.sync_copy(rows_t, acc.at[didx_t], add=True)
        if with_deg:
            pltpu.sync_copy(ones_v.at[0:TAIL, :], dega.at[didx_t], add=True)

    plsc.subcore_barrier()

    obase = c * N + s * RPT
    pltpu.sync_copy(acc.at[pl.ds(s * RPT, RPT), :],
                    out_hbm.at[pl.ds(obase, RPT), :])
    if with_deg:
        pltpu.sync_copy(dega.at[pl.ds(s * RPT, RPT), :],
                        deg_hbm.at[pl.ds(obase, RPT), :])


@functools.lru_cache(maxsize=None)
def _make_agg(with_deg):
    mesh = plsc.VectorSubcoreMesh(core_axis_name="c", subcore_axis_name="s")
    out_type = [jax.ShapeDtypeStruct((NC * N, D), jnp.float32)]
    scratch = [
        pltpu.VMEM_SHARED((N, D), jnp.float32),
    ]
    if with_deg:
        out_type.append(jax.ShapeDtypeStruct((NC * N, 16), jnp.float32))
        scratch.append(pltpu.VMEM_SHARED((N, 16), jnp.float32))
    scratch += [
        pltpu.VMEM((K,), jnp.int32),
        pltpu.VMEM((K,), jnp.int32),
        pltpu.VMEM((K, D), jnp.float32),
        pltpu.VMEM((TAIL,), jnp.int32),
        pltpu.VMEM((TAIL,), jnp.int32),
        pltpu.VMEM((TAIL, D), jnp.float32),
        pltpu.VMEM((ZR, D), jnp.float32),
    ]
    if with_deg:
        scratch += [
            pltpu.VMEM((ZR, 16), jnp.float32),
            pltpu.VMEM((K, 16), jnp.float32),
            pltpu.VMEM((16,), jnp.float32),
        ]
    scratch.append(pltpu.SemaphoreType.DMA)
    return pl.kernel(
        functools.partial(_agg_body, with_deg),
        out_type=tuple(out_type),
        mesh=mesh,
        scratch_types=scratch,
    )


def _pool_body(h0_hbm, h1_hbm, h2_hbm, h3_hbm, batch_hbm, out_hbm, cnt_hbm,
               a0, a1, a2, a3, cacc, bidx, rbuf, bidx_t, rbuf_t,
               zbuf, zbufc, ones_v, sem):
    c = lax.axis_index("c")
    s = lax.axis_index("s")
    accs = (a0, a1, a2, a3)
    hs = (h0_hbm, h1_hbm, h2_hbm, h3_hbm)

    _zero_vmem(zbuf, GPT, D)
    _zero_vmem(zbufc, GPT, 16)
    for r in range(PK):
        ones_v[r, :] = jnp.full((16,), 1.0, jnp.float32)

    gbase = s * GPT
    for a in accs:
        pltpu.sync_copy(zbuf, a.at[pl.ds(gbase, GPT), :])
    pltpu.sync_copy(zbufc, cacc.at[pl.ds(gbase, GPT), :])
    plsc.subcore_barrier()

    nbase = c * (N // NC) + s * PR_T

    @pl.loop(0, PNC)
    def _(i):
        off = pl.multiple_of(nbase + i * PK, 8)
        pltpu.sync_copy(batch_hbm.at[pl.ds(off, PK)], bidx)
        for h_hbm, a in zip(hs, accs):
            pltpu.sync_copy(h_hbm.at[pl.ds(off, PK), :], rbuf)
            pltpu.sync_copy(rbuf, a.at[bidx], add=True)
        pltpu.sync_copy(ones_v, cacc.at[bidx], add=True)

    if PREM:
        @pl.when(s == NS - 1)
        def _():
            off = pl.multiple_of(c * (N // NC) + NS * PR_T, 8)
            pltpu.sync_copy(batch_hbm.at[pl.ds(off, PREM)], bidx_t)
            for h_hbm, a in zip(hs, accs):
                pltpu.sync_copy(h_hbm.at[pl.ds(off, PREM), :], rbuf_t)
                pltpu.sync_copy(rbuf_t, a.at[bidx_t], add=True)
            pltpu.sync_copy(ones_v.at[0:PREM, :], cacc.at[bidx_t], add=True)

    plsc.subcore_barrier()

    for l, a in enumerate(accs):
        obase = (c * L + l) * G + gbase
        pltpu.sync_copy(a.at[pl.ds(gbase, GPT), :],
                        out_hbm.at[pl.ds(obase, GPT), :])
    pltpu.sync_copy(cacc.at[pl.ds(gbase, GPT), :],
                    cnt_hbm.at[pl.ds(c * G + gbase, GPT), :])


@functools.lru_cache(maxsize=None)
def _make_pool():
    mesh = plsc.VectorSubcoreMesh(core_axis_name="c", subcore_axis_name="s")
    return pl.kernel(
        _pool_body,
        out_type=(jax.ShapeDtypeStruct((NC * L * G, D), jnp.float32),
                  jax.ShapeDtypeStruct((NC * G, 16), jnp.float32)),
        mesh=mesh,
        scratch_types=[
            pltpu.VMEM_SHARED((G, D), jnp.float32),
            pltpu.VMEM_SHARED((G, D), jnp.float32),
            pltpu.VMEM_SHARED((G, D), jnp.float32),
            pltpu.VMEM_SHARED((G, D), jnp.float32),
            pltpu.VMEM_SHARED((G, 16), jnp.float32),
            pltpu.VMEM((PK,), jnp.int32),
            pltpu.VMEM((PK, D), jnp.float32),
            pltpu.VMEM((PREM,), jnp.int32),
            pltpu.VMEM((PREM, D), jnp.float32),
            pltpu.VMEM((GPT, D), jnp.float32),
            pltpu.VMEM((GPT, 16), jnp.float32),
            pltpu.VMEM((PK, 16), jnp.float32),
            pltpu.SemaphoreType.DMA,
        ],
    )


def _layer_body(h_ref, a0_ref, a1_ref, d0_ref, d1_ref, w1_ref, b1_ref,
                g_ref, be_ref, w2_ref, b2_ref, o_ref):
    deg = jnp.maximum(d0_ref[:, 0:1] + d1_ref[:, 0:1], 1.0)
    z = h_ref[...] + (a0_ref[...] + a1_ref[...]) / deg
    z = jnp.dot(z, w1_ref[...], preferred_element_type=jnp.float32) + b1_ref[...]
    mu = jnp.mean(z, axis=0, keepdims=True)
    var = jnp.mean(jnp.square(z - mu), axis=0, keepdims=True)
    z = (z - mu) * jax.lax.rsqrt(var + 1e-5) * g_ref[...] + be_ref[...]
    z = jnp.maximum(z, 0.0)
    z = jnp.dot(z, w2_ref[...], preferred_element_type=jnp.float32) + b2_ref[...]
    o_ref[...] = jnp.maximum(z, 0.0)


def _tc_layer(h, a0, a1, d0, d1, w1, b1, gamma, beta, w2, b2):
    return pl.pallas_call(
        _layer_body,
        out_shape=jax.ShapeDtypeStruct((N, H), jnp.float32),
    )(h, a0, a1, d0, d1, w1, b1, gamma, beta, w2, b2)


def _fc_body(pool_ref, cnt_ref, fcw_ref, fcb_ref, o_ref):
    cnt = jnp.maximum(cnt_ref[0:G, 0:1] + cnt_ref[G:2 * G, 0:1], 1.0)
    acc = jnp.zeros((G, C), jnp.float32) + fcb_ref[...]
    for l in range(L):
        p = (pool_ref[l * G:(l + 1) * G, :]
             + pool_ref[(L + l) * G:(L + l + 1) * G, :]) / cnt
        acc = acc + jnp.dot(p, fcw_ref[l * H:(l + 1) * H, :],
                            preferred_element_type=jnp.float32)
    m = jnp.max(acc, axis=-1, keepdims=True)
    sh = acc - m
    o_ref[...] = sh - jnp.log(jnp.sum(jnp.exp(sh), axis=-1, keepdims=True))


def _tc_fc(pool, cnt, fcw, fcb):
    return pl.pallas_call(
        _fc_body,
        out_shape=jax.ShapeDtypeStruct((G, C), jnp.float32),
    )(pool, cnt, fcw, fcb)


def kernel(x, edge_index, edge_attr, batch, W1, b1, gamma, beta, W2, b2,
           fcW, fcb):
    src = edge_index[0]
    dst = edge_index[1]
    agg_deg = _make_agg(True)
    agg = _make_agg(False)
    pool_k = _make_pool()

    h = x
    hs = []
    d0 = d1 = None
    for i in range(L):
        if i == 0:
            accf, degf = agg_deg(h, src, dst)
            d0, d1 = degf[:N], degf[N:]
        else:
            accf = agg(h, src, dst)
        h = _tc_layer(h, accf[:N], accf[N:], d0, d1,
                      W1[i], b1[i].reshape(1, 2 * H),
                      gamma[i].reshape(1, 2 * H), beta[i].reshape(1, 2 * H),
                      W2[i], b2[i].reshape(1, H))
        hs.append(h)

    pool, cnt = pool_k(hs[0], hs[1], hs[2], hs[3], batch)
    return _tc_fc(pool, cnt, fcW, fcb.reshape(1, C))


# speedup vs baseline: 5.2248x; 5.2248x over previous
import functools

import jax
import jax.numpy as jnp
from jax import lax
from jax.experimental import pallas as pl
from jax.experimental.pallas import tpu as pltpu
from jax.experimental.pallas import tpu_sc as plsc

N = 10000
E = 320000
D = 128
H = 128
L = 4
C = 32
G = 256

NC = 2
NS = 16
EPC = E // NC
EPT = EPC // NS
K = 128
NFULL = EPT // K
TAIL = EPT - NFULL * K
ZB = 624
ZR = 16
ZCNT = ZB // ZR

PR_T = (N // NC) // NS
PK = 104
PNC = PR_T // PK
PREM = N // NC - NS * PR_T
GPT = G // NS


def _zero_vmem(ref, rows):
    zero16 = jnp.zeros((16,), jnp.float32)
    for r in range(rows):
        for q in range(D // 16):
            ref[r, pl.ds(q * 16, 16)] = zero16


def _fill_ones(ref, rows):
    one16 = jnp.full((16,), 1.0, jnp.float32)
    for r in range(rows):
        for q in range(D // 16):
            ref[r, pl.ds(q * 16, 16)] = one16


def _agg_body(h_hbm, src_hbm, dst_hbm, out_hbm,
              acc, sidx, didx, rows, sidx_t, didx_t, rows_t, zbuf, sem):
    c = lax.axis_index("c")
    s = lax.axis_index("s")

    _zero_vmem(zbuf, ZR)
    rbase = s * ZB

    @pl.loop(0, ZCNT)
    def _(i):
        pltpu.sync_copy(zbuf, acc.at[pl.ds(rbase + i * ZR, ZR), :])

    @pl.when(s == NS - 1)
    def _():
        pltpu.sync_copy(zbuf, acc.at[pl.ds(N - ZR, ZR), :])

    plsc.subcore_barrier()

    ebase = c * EPC + s * EPT

    @pl.loop(0, NFULL)
    def _(i):
        off = pl.multiple_of(ebase + i * K, 8)
        pltpu.sync_copy(src_hbm.at[pl.ds(off, K)], sidx)
        pltpu.sync_copy(dst_hbm.at[pl.ds(off, K)], didx)
        pltpu.async_copy(h_hbm.at[sidx], rows, sem).wait()
        pltpu.sync_copy(rows, acc.at[didx], add=True)

    off = pl.multiple_of(ebase + NFULL * K, 8)
    pltpu.sync_copy(src_hbm.at[pl.ds(off, TAIL)], sidx_t)
    pltpu.sync_copy(dst_hbm.at[pl.ds(off, TAIL)], didx_t)
    pltpu.async_copy(h_hbm.at[sidx_t], rows_t, sem).wait()
    pltpu.sync_copy(rows_t, acc.at[didx_t], add=True)

    plsc.subcore_barrier()

    obase = c * N + rbase
    pltpu.sync_copy(acc.at[pl.ds(rbase, ZB), :],
                    out_hbm.at[pl.ds(obase, ZB), :])

    @pl.when(s == NS - 1)
    def _():
        pltpu.sync_copy(acc.at[pl.ds(N - ZR, ZR), :],
                        out_hbm.at[pl.ds(c * N + N - ZR, ZR), :])


@functools.lru_cache(maxsize=None)
def _make_agg():
    mesh = plsc.VectorSubcoreMesh(core_axis_name="c", subcore_axis_name="s")
    return pl.kernel(
        _agg_body,
        out_type=jax.ShapeDtypeStruct((NC * N, D), jnp.float32),
        mesh=mesh,
        scratch_types=[
            pltpu.VMEM_SHARED((N, D), jnp.float32),
            pltpu.VMEM((K,), jnp.int32),
            pltpu.VMEM((K,), jnp.int32),
            pltpu.VMEM((K, D), jnp.float32),
            pltpu.VMEM((TAIL,), jnp.int32),
            pltpu.VMEM((TAIL,), jnp.int32),
            pltpu.VMEM((TAIL, D), jnp.float32),
            pltpu.VMEM((ZR, D), jnp.float32),
            pltpu.SemaphoreType.DMA,
        ],
    )


def _deg_body(dst_hbm, out_hbm, acc, didx, didx_t, ones_v, zbuf, sem):
    c = lax.axis_index("c")
    s = lax.axis_index("s")

    _zero_vmem(zbuf, ZR)
    _fill_ones(ones_v, K)
    rbase = s * ZB

    @pl.loop(0, ZCNT)
    def _(i):
        pltpu.sync_copy(zbuf, acc.at[pl.ds(rbase + i * ZR, ZR), :])

    @pl.when(s == NS - 1)
    def _():
        pltpu.sync_copy(zbuf, acc.at[pl.ds(N - ZR, ZR), :])

    plsc.subcore_barrier()

    ebase = c * EPC + s * EPT

    @pl.loop(0, NFULL)
    def _(i):
        off = pl.multiple_of(ebase + i * K, 8)
        pltpu.sync_copy(dst_hbm.at[pl.ds(off, K)], didx)
        pltpu.sync_copy(ones_v, acc.at[didx], add=True)

    off = pl.multiple_of(ebase + NFULL * K, 8)
    pltpu.sync_copy(dst_hbm.at[pl.ds(off, TAIL)], didx_t)
    pltpu.sync_copy(ones_v.at[pl.ds(0, TAIL), :], acc.at[didx_t], add=True)

    plsc.subcore_barrier()

    obase = c * N + rbase
    pltpu.sync_copy(acc.at[pl.ds(rbase, ZB), :],
                    out_hbm.at[pl.ds(obase, ZB), :])

    @pl.when(s == NS - 1)
    def _():
        pltpu.sync_copy(acc.at[pl.ds(N - ZR, ZR), :],
                        out_hbm.at[pl.ds(c * N + N - ZR, ZR), :])


@functools.lru_cache(maxsize=None)
def _make_deg():
    mesh = plsc.VectorSubcoreMesh(core_axis_name="c", subcore_axis_name="s")
    return pl.kernel(
        _deg_body,
        out_type=jax.ShapeDtypeStruct((NC * N, D), jnp.float32),
        mesh=mesh,
        scratch_types=[
            pltpu.VMEM_SHARED((N, D), jnp.float32),
            pltpu.VMEM((K,), jnp.int32),
            pltpu.VMEM((TAIL,), jnp.int32),
            pltpu.VMEM((K, D), jnp.float32),
            pltpu.VMEM((ZR, D), jnp.float32),
            pltpu.SemaphoreType.DMA,
        ],
    )


def _pool_body(h0_hbm, h1_hbm, h2_hbm, h3_hbm, batch_hbm, out_hbm, cnt_hbm,
               a0, a1, a2, a3, cacc, bidx, rbuf, bidx_t, rbuf_t,
               zbuf, ones_v, sem):
    c = lax.axis_index("c")
    s = lax.axis_index("s")
    accs = (a0, a1, a2, a3)
    hs = (h0_hbm, h1_hbm, h2_hbm, h3_hbm)

    _zero_vmem(zbuf, GPT)
    _fill_ones(ones_v, PK)

    gbase = s * GPT
    for a in accs:
        pltpu.sync_copy(zbuf, a.at[pl.ds(gbase, GPT), :])
    pltpu.sync_copy(zbuf, cacc.at[pl.ds(gbase, GPT), :])
    plsc.subcore_barrier()

    nbase = c * (N // NC) + s * PR_T

    @pl.loop(0, PNC)
    def _(i):
        off = pl.multiple_of(nbase + i * PK, 8)
        pltpu.sync_copy(batch_hbm.at[pl.ds(off, PK)], bidx)
        for h_hbm, a in zip(hs, accs):
            pltpu.sync_copy(h_hbm.at[pl.ds(off, PK), :], rbuf)
            pltpu.sync_copy(rbuf, a.at[bidx], add=True)
        pltpu.sync_copy(ones_v, cacc.at[bidx], add=True)

    @pl.when(s == NS - 1)
    def _():
        off = pl.multiple_of(c * (N // NC) + NS * PR_T, 8)
        pltpu.sync_copy(batch_hbm.at[pl.ds(off, PREM)], bidx_t)
        for h_hbm, a in zip(hs, accs):
            pltpu.sync_copy(h_hbm.at[pl.ds(off, PREM), :], rbuf_t)
            pltpu.sync_copy(rbuf_t, a.at[bidx_t], add=True)
        pltpu.sync_copy(ones_v.at[pl.ds(0, PREM), :], cacc.at[bidx_t],
                        add=True)

    plsc.subcore_barrier()

    for l, a in enumerate(accs):
        obase = (c * L + l) * G + gbase
        pltpu.sync_copy(a.at[pl.ds(gbase, GPT), :],
                        out_hbm.at[pl.ds(obase, GPT), :])
    pltpu.sync_copy(cacc.at[pl.ds(gbase, GPT), :],
                    cnt_hbm.at[pl.ds(c * G + gbase, GPT), :])


@functools.lru_cache(maxsize=None)
def _make_pool():
    mesh = plsc.VectorSubcoreMesh(core_axis_name="c", subcore_axis_name="s")
    return pl.kernel(
        _pool_body,
        out_type=(jax.ShapeDtypeStruct((NC * L * G, D), jnp.float32),
                  jax.ShapeDtypeStruct((NC * G, D), jnp.float32)),
        mesh=mesh,
        scratch_types=[
            pltpu.VMEM_SHARED((G, D), jnp.float32),
            pltpu.VMEM_SHARED((G, D), jnp.float32),
            pltpu.VMEM_SHARED((G, D), jnp.float32),
            pltpu.VMEM_SHARED((G, D), jnp.float32),
            pltpu.VMEM_SHARED((G, D), jnp.float32),
            pltpu.VMEM((PK,), jnp.int32),
            pltpu.VMEM((PK, D), jnp.float32),
            pltpu.VMEM((PREM,), jnp.int32),
            pltpu.VMEM((PREM, D), jnp.float32),
            pltpu.VMEM((GPT, D), jnp.float32),
            pltpu.VMEM((PK, D), jnp.float32),
            pltpu.SemaphoreType.DMA,
        ],
    )


def _layer_body(h_ref, a0_ref, a1_ref, dinv_ref, w1_ref, b1_ref,
                g_ref, be_ref, w2_ref, b2_ref, o_ref):
    z = h_ref[...] + (a0_ref[...] + a1_ref[...]) * dinv_ref[...]
    z = jnp.dot(z, w1_ref[...], preferred_element_type=jnp.float32) + b1_ref[...]
    mu = jnp.mean(z, axis=0, keepdims=True)
    var = jnp.mean(jnp.square(z - mu), axis=0, keepdims=True)
    z = (z - mu) * jax.lax.rsqrt(var + 1e-5) * g_ref[...] + be_ref[...]
    z = jnp.maximum(z, 0.0)
    z = jnp.dot(z, w2_ref[...], preferred_element_type=jnp.float32) + b2_ref[...]
    o_ref[...] = jnp.maximum(z, 0.0)


def _tc_layer(h, a0, a1, dinv, w1, b1, gamma, beta, w2, b2):
    return pl.pallas_call(
        _layer_body,
        out_shape=jax.ShapeDtypeStruct((N, H), jnp.float32),
    )(h, a0, a1, dinv, w1, b1, gamma, beta, w2, b2)


def _dinv_body(d0_ref, d1_ref, o_ref):
    o_ref[...] = 1.0 / jnp.maximum(d0_ref[...] + d1_ref[...], 1.0)


def _tc_dinv(d0, d1):
    return pl.pallas_call(
        _dinv_body,
        out_shape=jax.ShapeDtypeStruct((N, 1), jnp.float32),
    )(d0, d1)


def _fc_body(pool_ref, cnt_ref, fcw_ref, fcb_ref, o_ref):
    cnt = jnp.maximum(cnt_ref[0:G, 0:1] + cnt_ref[G:2 * G, 0:1], 1.0)
    acc = jnp.zeros((G, C), jnp.float32) + fcb_ref[...]
    for l in range(L):
        p = (pool_ref[l * G:(l + 1) * G, :]
             + pool_ref[(L + l) * G:(L + l + 1) * G, :]) / cnt
        acc = acc + jnp.dot(p, fcw_ref[l * H:(l + 1) * H, :],
                            preferred_element_type=jnp.float32)
    m = jnp.max(acc, axis=-1, keepdims=True)
    sh = acc - m
    o_ref[...] = sh - jnp.log(jnp.sum(jnp.exp(sh), axis=-1, keepdims=True))


def _tc_fc(pool, cnt, fcw, fcb):
    return pl.pallas_call(
        _fc_body,
        out_shape=jax.ShapeDtypeStruct((G, C), jnp.float32),
    )(pool, cnt, fcw, fcb)


def kernel(x, edge_index, edge_attr, batch, W1, b1, gamma, beta, W2, b2,
           fcW, fcb):
    src = edge_index[0]
    dst = edge_index[1]
    agg = _make_agg()
    deg_k = _make_deg()
    pool_k = _make_pool()

    degf = deg_k(dst)
    dinv = _tc_dinv(degf[:N, 0:1], degf[N:, 0:1])

    h = x
    hs = []
    for i in range(L):
        accf = agg(h, src, dst)
        h = _tc_layer(h, accf[:N], accf[N:], dinv,
                      W1[i], b1[i].reshape(1, 2 * H),
                      gamma[i].reshape(1, 2 * H), beta[i].reshape(1, 2 * H),
                      W2[i], b2[i].reshape(1, H))
        hs.append(h)

    pool, cnt = pool_k(hs[0], hs[1], hs[2], hs[3], batch)
    return _tc_fc(pool, cnt, fcW, fcb.reshape(1, C))
